# baseline (speedup 1.0000x reference)
import functools

import jax
import jax.numpy as jnp
from jax import lax
from jax.experimental import pallas as pl
from jax.experimental.pallas import tpu as pltpu
from jax.experimental.pallas import tpu_sc as plsc

NUM_CORES = 2
NUM_SUBCORES = 16
LANES = 16
NUM_WORKERS = NUM_CORES * NUM_SUBCORES

BATCH = 16384
DIM = 32
CHUNK = BATCH // NUM_WORKERS
R = 8
NPAIR = CHUNK // (2 * R)


def _body(user_hbm, item_hbm, ue_hbm, ie_hbm, ub_hbm, ib_hbm, off_hbm,
          out_hbm, uidx_v, iidx_v, blk_v, ub_v, ib_v, out_v, off_v, sem):
    wid = lax.axis_index("s") * NUM_CORES + lax.axis_index("c")
    base = wid * CHUNK

    for j in range(CHUNK // 128):
        pltpu.sync_copy(user_hbm.at[pl.ds(base + j * 128, 128)], uidx_v.at[j])
        pltpu.sync_copy(item_hbm.at[pl.ds(base + j * 128, 128)], iidx_v.at[j])
    pltpu.sync_copy(off_hbm, off_v.at[pl.ds(0, 1)])
    off = off_v[pl.ds(0, LANES)][0]

    iv = lax.iota(jnp.int32, LANES)
    ivu = iv % R
    ivi = ivu + R
    himask = iv >= R

    def round_acc(p, half):
        rr = 2 * p + half
        row = rr // (128 // R)
        col0 = (rr % (128 // R)) * R
        rowvec = jnp.full((LANES,), row, jnp.int32)
        ru = plsc.load_gather(uidx_v, (rowvec, col0 + ivu))
        ri = plsc.load_gather(iidx_v, (rowvec, col0 + ivu))
        lane_u = ru % 128
        lane_i = ri % 128
        copies = []
        for i in range(R):
            su = pl.multiple_of((ru[i] // 128) * 128, 128)
            si = pl.multiple_of((ri[i] // 128) * 128, 128)
            copies.append(pltpu.async_copy(
                ue_hbm.at[:, pl.ds(su, 128)], blk_v.at[i], sem))
            copies.append(pltpu.async_copy(
                ie_hbm.at[:, pl.ds(si, 128)], blk_v.at[R + i], sem))
        bu = pltpu.async_copy(
            ub_hbm.at[uidx_v.at[row].at[pl.ds(col0, R)]], ub_v, sem)
        bi = pltpu.async_copy(
            ib_hbm.at[iidx_v.at[row].at[pl.ds(col0, R)]], ib_v, sem)
        for c in copies:
            c.wait()
        bu.wait()
        bi.wait()
        acc = (plsc.load_gather(ub_v, (ivu,)) + plsc.load_gather(ib_v, (ivu,))
               + off)
        for d in range(DIM):
            dvec = jnp.full((LANES,), d, jnp.int32)
            gu = plsc.load_gather(blk_v, (ivu, dvec, lane_u))
            gi = plsc.load_gather(blk_v, (ivi, dvec, lane_i))
            acc = acc + gu * gi
        return acc

    def pair(p, carry):
        acc_a = round_acc(p, 0)
        acc_b = round_acc(p, 1)
        acc = jnp.where(himask, acc_b, acc_a)
        out_v[pl.ds(p * LANES, LANES)] = 5.5 / (1.0 + jnp.exp(-acc))
        return carry

    lax.fori_loop(0, NPAIR, pair, 0)
    pltpu.sync_copy(out_v, out_hbm.at[pl.ds(base, CHUNK)])


@jax.jit
def kernel(user, item, user_emb, item_emb, user_bias, item_bias, offset):
    run = functools.partial(
        pl.kernel,
        out_type=jax.ShapeDtypeStruct((BATCH,), jnp.float32),
        mesh=plsc.VectorSubcoreMesh(core_axis_name="c", subcore_axis_name="s"),
        scratch_types=[
            pltpu.VMEM((CHUNK // 128, 128), jnp.int32),
            pltpu.VMEM((CHUNK // 128, 128), jnp.int32),
            pltpu.VMEM((2 * R, DIM, 128), jnp.float32),
            pltpu.VMEM((R,), jnp.float32),
            pltpu.VMEM((R,), jnp.float32),
            pltpu.VMEM((CHUNK,), jnp.float32),
            pltpu.VMEM((LANES,), jnp.float32),
            pltpu.SemaphoreType.DMA,
        ],
        compiler_params=pltpu.CompilerParams(
            needs_layout_passes=False, use_tc_tiling_on_sc=True),
    )(_body)
    return run(user.astype(jnp.int32), item.astype(jnp.int32),
               user_emb.T, item_emb.T, user_bias, item_bias, offset)
